# Initial kernel scaffold; baseline (speedup 1.0000x reference)
import jax
import jax.numpy as jnp
from jax.experimental import pallas as pl


def kernel(texts, table, W, b):
    raise NotImplementedError("write your pallas kernel here")


# speedup vs baseline: 21.0956x; 21.0956x over previous
import functools

import jax
import jax.numpy as jnp
from jax import lax
from jax.experimental import pallas as pl
from jax.experimental.pallas import tpu as pltpu
from jax.experimental.pallas import tpu_sc as plsc

_VOCAB = 100000
_DIM = 128
_BATCH = 4096
_SEQ = 200
_NCLS = 2

_TC_BLK = 2000


def _tc_project_body(t_ref, w_ref, b_ref, o_ref):
    o_ref[...] = (
        jnp.dot(t_ref[...], w_ref[...], preferred_element_type=jnp.float32)
        + b_ref[...]
    )


def _tc_project(table, W, b):
    grid = _VOCAB // _TC_BLK
    return pl.pallas_call(
        _tc_project_body,
        grid=(grid,),
        in_specs=[
            pl.BlockSpec((_TC_BLK, _DIM), lambda i: (i, 0)),
            pl.BlockSpec((_DIM, _NCLS), lambda i: (0, 0)),
            pl.BlockSpec((1, _NCLS), lambda i: (0, 0)),
        ],
        out_specs=pl.BlockSpec((_TC_BLK, _NCLS), lambda i: (i, 0)),
        out_shape=jax.ShapeDtypeStruct((_VOCAB, _NCLS), jnp.float32),
    )(table, W, b.reshape(1, _NCLS))


_ROWS_PER_TILE = _BATCH // 16
_CHUNK_ROWS = 64
_N_CHUNKS = _ROWS_PER_TILE // _CHUNK_ROWS
_N_GROUPS = _CHUNK_ROWS // 16


@functools.partial(
    pl.kernel,
    out_type=jax.ShapeDtypeStruct((_NCLS, _BATCH), jnp.float32),
    mesh=plsc.VectorSubcoreMesh(core_axis_name="c", subcore_axis_name="s"),
    compiler_params=pltpu.CompilerParams(needs_layout_passes=False),
    scratch_types=[
        pltpu.VMEM((_VOCAB,), jnp.float32),
        pltpu.VMEM((_CHUNK_ROWS * _SEQ,), jnp.int32),
        pltpu.VMEM((_ROWS_PER_TILE,), jnp.float32),
    ],
)
def _sc_pool(pt_hbm, texts_hbm, out_hbm, pcol, tchunk, obuf):
    cls = lax.axis_index("c")
    sub = lax.axis_index("s")
    r0 = sub * _ROWS_PER_TILE

    pltpu.sync_copy(pt_hbm.at[cls], pcol)

    lane_off = lax.iota(jnp.int32, 16) * _SEQ
    inv_l = jnp.float32(1.0 / _SEQ)
    zero = jnp.zeros((16,), jnp.float32)

    for ch in range(_N_CHUNKS):
        pltpu.sync_copy(
            texts_hbm.at[
                pl.ds((r0 + ch * _CHUNK_ROWS) * _SEQ, _CHUNK_ROWS * _SEQ)
            ],
            tchunk,
        )

        def body(t, accs):
            out = []
            for g in range(_N_GROUPS):
                idx = lane_off + (t + g * (16 * _SEQ))
                tok = plsc.load_gather(tchunk, [idx])
                val = plsc.load_gather(pcol, [tok])
                out.append(accs[g] + val)
            return tuple(out)

        accs = lax.fori_loop(0, _SEQ, body, (zero,) * _N_GROUPS)
        for g in range(_N_GROUPS):
            obuf[pl.ds(ch * _CHUNK_ROWS + g * 16, 16)] = accs[g] * inv_l

    pltpu.sync_copy(obuf, out_hbm.at[cls, pl.ds(r0, _ROWS_PER_TILE)])


def kernel(texts, table, W, b):
    texts_flat = texts.astype(jnp.int32).reshape(_BATCH * _SEQ)
    p = _tc_project(table, W, b)
    out_t = _sc_pool(p.T, texts_flat)
    return out_t.T


# speedup vs baseline: 31.5803x; 1.4970x over previous
import functools

import jax
import jax.numpy as jnp
from jax import lax
from jax.experimental import pallas as pl
from jax.experimental.pallas import tpu as pltpu
from jax.experimental.pallas import tpu_sc as plsc

_VOCAB = 100000
_DIM = 128
_BATCH = 4096
_SEQ = 200
_NCLS = 2

_TC_BLK = 2048
_TC_GRID = -(-_VOCAB // _TC_BLK)
_VOCAB_PAD = _TC_GRID * _TC_BLK

def _tc_project_body(t_ref, wt_ref, b_ref, o_ref):
    o_ref[...] = (
        lax.dot_general(
            wt_ref[...],
            t_ref[...],
            (((1,), (1,)), ((), ())),
            preferred_element_type=jnp.float32,
        )
        + b_ref[...]
    )


def _tc_project(table, W, b):
    return pl.pallas_call(
        _tc_project_body,
        grid=(_TC_GRID,),
        in_specs=[
            pl.BlockSpec((_TC_BLK, _DIM), lambda i: (i, 0)),
            pl.BlockSpec((_NCLS, _DIM), lambda i: (0, 0)),
            pl.BlockSpec((_NCLS, 1), lambda i: (0, 0)),
        ],
        out_specs=pl.BlockSpec((_NCLS, _TC_BLK), lambda i: (0, i)),
        out_shape=jax.ShapeDtypeStruct((_NCLS, _VOCAB_PAD), jnp.float32),
    )(table, W.T, b.reshape(_NCLS, 1))


_ROWS_PER_TILE = _BATCH // 16
_CHUNK_ROWS = 64
_N_CHUNKS = _ROWS_PER_TILE // _CHUNK_ROWS
_N_GROUPS = _CHUNK_ROWS // 16


@functools.partial(
    pl.kernel,
    out_type=jax.ShapeDtypeStruct((_NCLS, _BATCH), jnp.float32),
    mesh=plsc.VectorSubcoreMesh(core_axis_name="c", subcore_axis_name="s"),
    compiler_params=pltpu.CompilerParams(needs_layout_passes=False),
    scratch_types=[
        pltpu.VMEM((_VOCAB_PAD,), jnp.float32),
        pltpu.VMEM((_CHUNK_ROWS * _SEQ,), jnp.int32),
        pltpu.VMEM((_ROWS_PER_TILE,), jnp.float32),
    ],
)
def _sc_pool(pt_hbm, texts_hbm, out_hbm, pcol, tchunk, obuf):
    cls = lax.axis_index("c")
    sub = lax.axis_index("s")
    r0 = sub * _ROWS_PER_TILE

    pltpu.sync_copy(pt_hbm.at[cls], pcol)

    lane_off = lax.iota(jnp.int32, 16) * _SEQ
    inv_l = jnp.float32(1.0 / _SEQ)
    zero = jnp.zeros((16,), jnp.float32)

    for ch in range(_N_CHUNKS):
        pltpu.sync_copy(
            texts_hbm.at[
                pl.ds((r0 + ch * _CHUNK_ROWS) * _SEQ, _CHUNK_ROWS * _SEQ)
            ],
            tchunk,
        )

        def body(t, accs):
            out = []
            for g in range(_N_GROUPS):
                idx = lane_off + (t + g * (16 * _SEQ))
                tok = plsc.load_gather(tchunk, [idx])
                val = plsc.load_gather(pcol, [tok])
                out.append(accs[g] + val)
            return tuple(out)

        accs = lax.fori_loop(0, _SEQ, body, (zero,) * _N_GROUPS)
        for g in range(_N_GROUPS):
            obuf[pl.ds(ch * _CHUNK_ROWS + g * 16, 16)] = accs[g] * inv_l

    pltpu.sync_copy(obuf, out_hbm.at[cls, pl.ds(r0, _ROWS_PER_TILE)])


def kernel(texts, table, W, b):
    texts_flat = texts.astype(jnp.int32).reshape(_BATCH * _SEQ)
    pt = _tc_project(table, W, b)
    out_t = _sc_pool(pt, texts_flat)
    return out_t.T
